# Initial kernel scaffold; baseline (speedup 1.0000x reference)
import jax
import jax.numpy as jnp
from jax.experimental import pallas as pl


def kernel(positions, Z, neighbors, neighbor_mask, atom_mask, embedding, W_f1, b_f1, W_f2, b_f2, W_in, b_in, W_out, b_out, W_o1, b_o1, W_o2, b_o2):
    raise NotImplementedError("write your pallas kernel here")


# speedup vs baseline: 1317.8023x; 1317.8023x over previous
import jax
import jax.numpy as jnp
from jax.experimental import pallas as pl

_B, _A, _NBR, _F, _NRBF, _NDOS, _NINT = 32, 128, 64, 128, 50, 200, 3
_CUTOFF = 5.0
_GAMMA = 10.0
_NZ = 100
_E = _A * _NBR
_LOG2 = 0.6931471805599453


def _ssp(x):
    return jnp.maximum(x, 0.0) + jnp.log1p(jnp.exp(-jnp.abs(x))) - _LOG2


def _schnet_body(pos_ref, z_ref, nbr_ref, amask_ref, emb_ref,
                 wf1_ref, bf1_ref, wf2_ref, bf2_ref,
                 win_ref, bin_ref, wout_ref, bout_ref,
                 wo1_ref, bo1_ref, wo2_ref, bo2_ref, out_ref):
    f32 = jnp.float32
    pos = pos_ref[0]
    z = z_ref[0]
    nbr = nbr_ref[0]
    amask = amask_ref[0]

    zoh = (jax.lax.broadcasted_iota(jnp.int32, (_A, _NZ), 1) == z).astype(f32)
    x = jnp.dot(zoh, emb_ref[...], preferred_element_type=f32)

    col = jax.lax.broadcasted_iota(jnp.int32, (_E, _A), 1)
    row = jax.lax.broadcasted_iota(jnp.int32, (_E, _A), 0)
    og = (col == nbr).astype(f32)
    ot = (col == row // _NBR).astype(f32)

    r = jnp.dot(og - ot, pos, preferred_element_type=f32)
    d = jnp.sqrt(jnp.sum(r * r, axis=1, keepdims=True) + 1e-10)

    mu = jax.lax.broadcasted_iota(jnp.int32, (_E, _NRBF), 1).astype(f32)
    mu = mu * (_CUTOFF / (_NRBF - 1))
    rbf = jnp.exp(-_GAMMA * (d - mu) ** 2)
    h1 = _ssp(jnp.dot(rbf, wf1_ref[...], preferred_element_type=f32)
              + bf1_ref[...])
    wf = jnp.dot(h1, wf2_ref[...], preferred_element_type=f32) + bf2_ref[...]

    fcut = 0.5 * (jnp.cos(d * (jnp.pi / _CUTOFF)) + 1.0)
    fcut = fcut * (d < _CUTOFF).astype(f32)
    og = og * fcut

    for t in range(_NINT):
        v = jnp.dot(x, win_ref[t], preferred_element_type=f32) + bin_ref[t:t + 1]
        vj = jnp.dot(og, v, preferred_element_type=f32)
        y = jax.lax.dot_general(ot, vj * wf, (((0,), (0,)), ((), ())),
                                preferred_element_type=f32)
        y = _ssp(jnp.dot(y, wout_ref[t], preferred_element_type=f32)
                 + bout_ref[t:t + 1])
        x = x + y

    h = _ssp(jnp.dot(x, wo1_ref[...], preferred_element_type=f32) + bo1_ref[...])
    dos = jnp.dot(h, wo2_ref[...], preferred_element_type=f32) + bo2_ref[...]
    out_ref[0] = jnp.dot(amask, dos, preferred_element_type=f32)


def kernel(positions, Z, neighbors, neighbor_mask, atom_mask, embedding,
           W_f1, b_f1, W_f2, b_f2, W_in, b_in, W_out, b_out,
           W_o1, b_o1, W_o2, b_o2):
    f32 = jnp.float32
    nbr = jnp.where(neighbor_mask > 0, neighbors.astype(jnp.int32), _A)
    nbr = nbr.reshape(_B, _E, 1)
    z3 = Z.astype(jnp.int32).reshape(_B, _A, 1)
    amask3 = atom_mask.astype(f32).reshape(_B, 1, _A)

    def fullspec(x):
        return pl.BlockSpec(x.shape, lambda b: (0,) * x.ndim)

    b_f1 = b_f1.reshape(1, _F)
    b_f2 = b_f2.reshape(1, _F)
    b_o1 = b_o1.reshape(1, 64)
    b_o2 = b_o2.reshape(1, _NDOS)
    weights = (embedding, W_f1, b_f1, W_f2, b_f2, W_in, b_in, W_out, b_out,
               W_o1, b_o1, W_o2, b_o2)

    out = pl.pallas_call(
        _schnet_body,
        grid=(_B,),
        in_specs=[
            pl.BlockSpec((1, _A, 3), lambda b: (b, 0, 0)),
            pl.BlockSpec((1, _A, 1), lambda b: (b, 0, 0)),
            pl.BlockSpec((1, _E, 1), lambda b: (b, 0, 0)),
            pl.BlockSpec((1, 1, _A), lambda b: (b, 0, 0)),
        ] + [fullspec(w) for w in weights],
        out_specs=pl.BlockSpec((1, 1, _NDOS), lambda b: (b, 0, 0)),
        out_shape=jax.ShapeDtypeStruct((_B, 1, _NDOS), f32),
    )(positions.astype(f32), z3, nbr, amask3, *weights)
    return out.reshape(_B, _NDOS)


# speedup vs baseline: 3900.2507x; 2.9597x over previous
import jax
import jax.numpy as jnp
from jax.experimental import pallas as pl

_B, _A, _NBR, _F, _NRBF, _NDOS, _NINT = 32, 128, 64, 128, 50, 200, 3
_CUTOFF = 5.0
_GAMMA = 10.0
_NZ = 100
_E = _A * _NBR
_LOG2 = 0.6931471805599453


def _ssp(x):
    return jnp.maximum(x, 0.0) + jnp.log1p(jnp.exp(-jnp.abs(x))) - _LOG2


def _lanefold(x, width):
    while x.shape[1] > width:
        half = x.shape[1] // 2
        x = x[:, :half] + x[:, half:]
    return x


def _schnet_body(post_ref, z_ref, nbr_ref, amask_ref, embt_ref,
                 wf1t_ref, bf1_ref, wf2t_ref, bf2_ref,
                 wint_ref, bin_ref, woutt_ref, bout_ref,
                 wo1t_ref, bo1_ref, wo2t_ref, bo2_ref, out_ref):
    f32 = jnp.float32
    post = post_ref[0]
    z = z_ref[0]
    nbr = nbr_ref[0]
    amask = amask_ref[0]

    zoht = (jax.lax.broadcasted_iota(jnp.int32, (_NZ, _A), 0) == z).astype(f32)
    xt = jnp.dot(embt_ref[...], zoht, preferred_element_type=f32)

    row = jax.lax.broadcasted_iota(jnp.int32, (_A, _E), 0)
    col = jax.lax.broadcasted_iota(jnp.int32, (_A, _E), 1)
    ogt = (row == nbr).astype(f32)
    ott = (row == col % _A).astype(f32)

    rt = jnp.dot(post, ogt - ott, preferred_element_type=f32)
    d = jnp.sqrt(jnp.sum(rt * rt, axis=0, keepdims=True) + 1e-10)

    mu = jax.lax.broadcasted_iota(jnp.int32, (_NRBF, _E), 0).astype(f32)
    mu = mu * (_CUTOFF / (_NRBF - 1))
    rbft = jnp.exp(-_GAMMA * (d - mu) ** 2)
    h1t = _ssp(jnp.dot(wf1t_ref[...], rbft, preferred_element_type=f32)
               + bf1_ref[...])
    wft = jnp.dot(wf2t_ref[...], h1t, preferred_element_type=f32) + bf2_ref[...]

    fcut = 0.5 * (jnp.cos(d * (jnp.pi / _CUTOFF)) + 1.0)
    fcut = fcut * (d < _CUTOFF).astype(f32)
    ogt = ogt * fcut

    for t in range(_NINT):
        vt = jnp.dot(wint_ref[t], xt, preferred_element_type=f32) + bin_ref[t]
        vjt = jnp.dot(vt, ogt, preferred_element_type=f32)
        yt = _lanefold(vjt * wft, _A)
        yt = _ssp(jnp.dot(woutt_ref[t], yt, preferred_element_type=f32)
                  + bout_ref[t])
        xt = xt + yt

    ht = _ssp(jnp.dot(wo1t_ref[...], xt, preferred_element_type=f32)
              + bo1_ref[...])
    dost = jnp.dot(wo2t_ref[...], ht, preferred_element_type=f32) + bo2_ref[...]
    out_ref[0] = jnp.sum(dost * amask, axis=1, keepdims=True)


def kernel(positions, Z, neighbors, neighbor_mask, atom_mask, embedding,
           W_f1, b_f1, W_f2, b_f2, W_in, b_in, W_out, b_out,
           W_o1, b_o1, W_o2, b_o2):
    f32 = jnp.float32
    nbr = jnp.where(neighbor_mask > 0, neighbors.astype(jnp.int32), _A)
    nbr = nbr.transpose(0, 2, 1).reshape(_B, 1, _E)
    z3 = Z.astype(jnp.int32).reshape(_B, 1, _A)
    amask3 = atom_mask.astype(f32).reshape(_B, 1, _A)
    post = positions.astype(f32).transpose(0, 2, 1)

    def fullspec(x):
        return pl.BlockSpec(x.shape, lambda b: (0,) * x.ndim)

    weights = (embedding.T, W_f1.T, b_f1.reshape(_F, 1),
               W_f2.T, b_f2.reshape(_F, 1),
               W_in.transpose(0, 2, 1), b_in.reshape(_NINT, _F, 1),
               W_out.transpose(0, 2, 1), b_out.reshape(_NINT, _F, 1),
               W_o1.T, b_o1.reshape(64, 1),
               W_o2.T, b_o2.reshape(_NDOS, 1))

    out = pl.pallas_call(
        _schnet_body,
        grid=(_B,),
        in_specs=[
            pl.BlockSpec((1, 3, _A), lambda b: (b, 0, 0)),
            pl.BlockSpec((1, 1, _A), lambda b: (b, 0, 0)),
            pl.BlockSpec((1, 1, _E), lambda b: (b, 0, 0)),
            pl.BlockSpec((1, 1, _A), lambda b: (b, 0, 0)),
        ] + [fullspec(w) for w in weights],
        out_specs=pl.BlockSpec((1, _NDOS, 1), lambda b: (b, 0, 0)),
        out_shape=jax.ShapeDtypeStruct((_B, _NDOS, 1), f32),
    )(post, z3, nbr, amask3, *weights)
    return out.reshape(_B, _NDOS)


# speedup vs baseline: 4035.7580x; 1.0347x over previous
import jax
import jax.numpy as jnp
from jax.experimental import pallas as pl

_B, _A, _NBR, _F, _NRBF, _NDOS, _NINT = 32, 128, 64, 128, 50, 200, 3
_CUTOFF = 5.0
_GAMMA = 10.0
_NZ = 100
_E = _A * _NBR
_LOG2 = 0.6931471805599453


def _ssp(x):
    return jnp.maximum(x, 0.0) + jnp.log1p(jnp.exp(-jnp.abs(x))) - _LOG2


def _lanefold(x, width):
    while x.shape[1] > width:
        half = x.shape[1] // 2
        x = x[:, :half] + x[:, half:]
    return x


def _schnet_body(ph_ref, plo_ref, z_ref, nbr_ref, amask_ref, embt_ref,
                 wf1t_ref, bf1_ref, wf2t_ref, bf2_ref,
                 wint_ref, bin_ref, woutt_ref, bout_ref,
                 wo1t_ref, bo1_ref, wo2t_ref, bo2_ref, out_ref):
    f32 = jnp.float32
    bf16 = jnp.bfloat16
    phi = ph_ref[0]
    plo = plo_ref[0]
    z = z_ref[0]
    nbr = nbr_ref[0]
    amask = amask_ref[0]

    zoht = (jax.lax.broadcasted_iota(jnp.int32, (_NZ, _A), 0) == z).astype(f32)
    xt = jnp.dot(embt_ref[...], zoht, preferred_element_type=f32)

    row = jax.lax.broadcasted_iota(jnp.int32, (_A, _E), 0)
    ogt = (row == nbr).astype(bf16)

    pos_j = (jnp.dot(phi, ogt, preferred_element_type=f32)
             + jnp.dot(plo, ogt, preferred_element_type=f32))
    post = phi.astype(f32) + plo.astype(f32)
    rt = pos_j - jnp.concatenate([post] * _NBR, axis=1)
    d = jnp.sqrt(jnp.sum(rt * rt, axis=0, keepdims=True) + 1e-10)

    mu = jax.lax.broadcasted_iota(jnp.int32, (_NRBF, _E), 0).astype(f32)
    mu = mu * (_CUTOFF / (_NRBF - 1))
    rbft = jnp.exp(-_GAMMA * (d - mu) ** 2)
    h1t = _ssp(jnp.dot(wf1t_ref[...], rbft, preferred_element_type=f32)
               + bf1_ref[...])
    wft = jnp.dot(wf2t_ref[...], h1t, preferred_element_type=f32) + bf2_ref[...]

    fcut = 0.5 * (jnp.cos(jnp.minimum(d, _CUTOFF) * (jnp.pi / _CUTOFF)) + 1.0)
    wft = wft * fcut

    for t in range(_NINT):
        vt = jnp.dot(wint_ref[t], xt, preferred_element_type=f32) + bin_ref[t]
        vjt = jnp.dot(vt.astype(bf16), ogt, preferred_element_type=f32)
        yt = _lanefold(vjt * wft, _A)
        yt = _ssp(jnp.dot(woutt_ref[t], yt, preferred_element_type=f32)
                  + bout_ref[t])
        xt = xt + yt

    ht = _ssp(jnp.dot(wo1t_ref[...], xt, preferred_element_type=f32)
              + bo1_ref[...])
    dost = jnp.dot(wo2t_ref[...], ht, preferred_element_type=f32) + bo2_ref[...]
    out_ref[0] = jnp.sum(dost * amask, axis=1, keepdims=True)


def kernel(positions, Z, neighbors, neighbor_mask, atom_mask, embedding,
           W_f1, b_f1, W_f2, b_f2, W_in, b_in, W_out, b_out,
           W_o1, b_o1, W_o2, b_o2):
    f32 = jnp.float32
    bf16 = jnp.bfloat16
    nbr = jnp.where(neighbor_mask > 0, neighbors.astype(jnp.int32), _A)
    nbr = nbr.transpose(0, 2, 1).reshape(_B, 1, _E)
    z3 = Z.astype(jnp.int32).reshape(_B, 1, _A)
    amask3 = atom_mask.astype(f32).reshape(_B, 1, _A)
    post = positions.astype(f32).transpose(0, 2, 1)
    post_hi = post.astype(bf16)
    post_lo = (post - post_hi.astype(f32)).astype(bf16)

    def fullspec(x):
        return pl.BlockSpec(x.shape, lambda b: (0,) * x.ndim)

    weights = (embedding.T, W_f1.T, b_f1.reshape(_F, 1),
               W_f2.T, b_f2.reshape(_F, 1),
               W_in.transpose(0, 2, 1), b_in.reshape(_NINT, _F, 1),
               W_out.transpose(0, 2, 1), b_out.reshape(_NINT, _F, 1),
               W_o1.T, b_o1.reshape(64, 1),
               W_o2.T, b_o2.reshape(_NDOS, 1))

    out = pl.pallas_call(
        _schnet_body,
        grid=(_B,),
        in_specs=[
            pl.BlockSpec((1, 3, _A), lambda b: (b, 0, 0)),
            pl.BlockSpec((1, 3, _A), lambda b: (b, 0, 0)),
            pl.BlockSpec((1, 1, _A), lambda b: (b, 0, 0)),
            pl.BlockSpec((1, 1, _E), lambda b: (b, 0, 0)),
            pl.BlockSpec((1, 1, _A), lambda b: (b, 0, 0)),
        ] + [fullspec(w) for w in weights],
        out_specs=pl.BlockSpec((1, _NDOS, 1), lambda b: (b, 0, 0)),
        out_shape=jax.ShapeDtypeStruct((_B, _NDOS, 1), f32),
    )(post_hi, post_lo, z3, nbr, amask3, *weights)
    return out.reshape(_B, _NDOS)


# speedup vs baseline: 4546.4478x; 1.1265x over previous
import jax
import jax.numpy as jnp
from jax.experimental import pallas as pl

_B, _A, _NBR, _F, _NRBF, _NDOS, _NINT = 32, 128, 64, 128, 50, 200, 3
_CUTOFF = 5.0
_GAMMA = 10.0
_NZ = 100
_E = _A * _NBR
_LOG2 = 0.6931471805599453


def _ssp(x):
    return jnp.maximum(x, 0.0) + jnp.log1p(jnp.exp(-jnp.abs(x))) - _LOG2


def _ssp_fast(x):
    return jnp.log1p(jnp.exp(x)) - _LOG2


_FCUT_COEF = (1.0, -0.25, 2.08333333e-02, -6.94444423e-04, 1.24007823e-05,
              -1.37783258e-07, 1.04325984e-09, -5.67677417e-12, 2.06565961e-14)


def _fcut_poly(u):
    acc = jnp.full_like(u, _FCUT_COEF[-1])
    for c in _FCUT_COEF[-2::-1]:
        acc = acc * u + c
    return acc


def _lanefold(x, width):
    while x.shape[1] > width:
        half = x.shape[1] // 2
        x = x[:, :half] + x[:, half:]
    return x


def _schnet_body(ph_ref, plo_ref, z_ref, nbr_ref, amask_ref, embt_ref,
                 wf1t_ref, bf1_ref, wf2t_ref, bf2_ref,
                 wint_ref, bin_ref, woutt_ref, bout_ref,
                 wo1t_ref, bo1_ref, wo2t_ref, bo2_ref, out_ref):
    f32 = jnp.float32
    bf16 = jnp.bfloat16
    phi = ph_ref[0]
    plo = plo_ref[0]
    z = z_ref[0]
    nbr = nbr_ref[0]
    amask = amask_ref[0]

    zoht = (jax.lax.broadcasted_iota(jnp.int32, (_NZ, _A), 0) == z).astype(f32)
    xt = jnp.dot(embt_ref[...], zoht, preferred_element_type=f32)

    row = jax.lax.broadcasted_iota(jnp.int32, (_A, _E), 0)
    ogt = (row == nbr).astype(bf16)

    pos_j = (jnp.dot(phi, ogt, preferred_element_type=f32)
             + jnp.dot(plo, ogt, preferred_element_type=f32))
    post = phi.astype(f32) + plo.astype(f32)
    rt = pos_j - jnp.concatenate([post] * _NBR, axis=1)
    d = jnp.sqrt(jnp.sum(rt * rt, axis=0, keepdims=True) + 1e-10)

    mu = jax.lax.broadcasted_iota(jnp.int32, (_NRBF, _E), 0).astype(f32)
    mu = mu * (_CUTOFF / (_NRBF - 1))
    rbft = jnp.exp(-_GAMMA * (d - mu) ** 2)
    h1t = _ssp_fast(jnp.dot(wf1t_ref[...], rbft, preferred_element_type=f32)
                    + bf1_ref[...])
    wft = jnp.dot(wf2t_ref[...], h1t, preferred_element_type=f32) + bf2_ref[...]

    theta = jnp.minimum(d, _CUTOFF) * (jnp.pi / _CUTOFF)
    wft = wft * _fcut_poly(theta * theta)

    for t in range(_NINT):
        vt = jnp.dot(wint_ref[t], xt, preferred_element_type=f32) + bin_ref[t]
        vjt = jnp.dot(vt.astype(bf16), ogt, preferred_element_type=f32)
        yt = _lanefold(vjt * wft, _A)
        yt = _ssp(jnp.dot(woutt_ref[t], yt, preferred_element_type=f32)
                  + bout_ref[t])
        xt = xt + yt

    ht = _ssp(jnp.dot(wo1t_ref[...], xt, preferred_element_type=f32)
              + bo1_ref[...])
    dost = jnp.dot(wo2t_ref[...], ht, preferred_element_type=f32) + bo2_ref[...]
    out_ref[0] = jnp.sum(dost * amask, axis=1, keepdims=True)


def kernel(positions, Z, neighbors, neighbor_mask, atom_mask, embedding,
           W_f1, b_f1, W_f2, b_f2, W_in, b_in, W_out, b_out,
           W_o1, b_o1, W_o2, b_o2):
    f32 = jnp.float32
    bf16 = jnp.bfloat16
    nbr = jnp.where(neighbor_mask > 0, neighbors.astype(jnp.int32), _A)
    nbr = nbr.transpose(0, 2, 1).reshape(_B, 1, _E)
    z3 = Z.astype(jnp.int32).reshape(_B, 1, _A)
    amask3 = atom_mask.astype(f32).reshape(_B, 1, _A)
    post = positions.astype(f32).transpose(0, 2, 1)
    post_hi = post.astype(bf16)
    post_lo = (post - post_hi.astype(f32)).astype(bf16)

    def fullspec(x):
        return pl.BlockSpec(x.shape, lambda b: (0,) * x.ndim)

    weights = (embedding.T, W_f1.T, b_f1.reshape(_F, 1),
               W_f2.T, b_f2.reshape(_F, 1),
               W_in.transpose(0, 2, 1), b_in.reshape(_NINT, _F, 1),
               W_out.transpose(0, 2, 1), b_out.reshape(_NINT, _F, 1),
               W_o1.T, b_o1.reshape(64, 1),
               W_o2.T, b_o2.reshape(_NDOS, 1))

    out = pl.pallas_call(
        _schnet_body,
        grid=(_B,),
        in_specs=[
            pl.BlockSpec((1, 3, _A), lambda b: (b, 0, 0)),
            pl.BlockSpec((1, 3, _A), lambda b: (b, 0, 0)),
            pl.BlockSpec((1, 1, _A), lambda b: (b, 0, 0)),
            pl.BlockSpec((1, 1, _E), lambda b: (b, 0, 0)),
            pl.BlockSpec((1, 1, _A), lambda b: (b, 0, 0)),
        ] + [fullspec(w) for w in weights],
        out_specs=pl.BlockSpec((1, _NDOS, 1), lambda b: (b, 0, 0)),
        out_shape=jax.ShapeDtypeStruct((_B, _NDOS, 1), f32),
    )(post_hi, post_lo, z3, nbr, amask3, *weights)
    return out.reshape(_B, _NDOS)


# speedup vs baseline: 4782.6712x; 1.0520x over previous
import jax
import jax.numpy as jnp
from jax.experimental import pallas as pl

_B, _A, _NBR, _F, _NRBF, _NDOS, _NINT = 32, 128, 64, 128, 50, 200, 3
_CUTOFF = 5.0
_GAMMA = 10.0
_NZ = 100
_E = _A * _NBR
_LOG2 = 0.6931471805599453


def _ssp(x):
    return jnp.maximum(x, 0.0) + jnp.log1p(jnp.exp(-jnp.abs(x))) - _LOG2


def _ssp_fast(x):
    return jnp.log1p(jnp.exp(x)) - _LOG2


_FCUT_COEF = (1.0, -0.25, 2.08333333e-02, -6.94444423e-04, 1.24007823e-05,
              -1.37783258e-07, 1.04325984e-09, -5.67677417e-12, 2.06565961e-14)


def _fcut_poly(u):
    acc = jnp.full_like(u, _FCUT_COEF[-1])
    for c in _FCUT_COEF[-2::-1]:
        acc = acc * u + c
    return acc


def _lanefold(x, width):
    while x.shape[1] > width:
        half = x.shape[1] // 2
        x = x[:, :half] + x[:, half:]
    return x


def _schnet_body(ph_ref, plo_ref, z_ref, nbr_ref, amask_ref, embt_ref,
                 wf1t_ref, bf1_ref, wf2t_ref, bf2_ref,
                 wint_ref, bin_ref, woutt_ref, bout_ref,
                 wo1t_ref, bo1_ref, wo2t_ref, bo2_ref, out_ref):
    f32 = jnp.float32
    bf16 = jnp.bfloat16
    phi = ph_ref[0]
    plo = plo_ref[0]
    z = z_ref[0]
    nbr = nbr_ref[0]
    amask = amask_ref[0]

    zoht = (jax.lax.broadcasted_iota(jnp.int32, (_NZ, 2 * _A), 0) == z).astype(f32)
    xt = jnp.dot(embt_ref[...], zoht, preferred_element_type=f32)

    row = jax.lax.broadcasted_iota(jnp.int32, (_A, 2 * _E), 0)
    ogt = (row == nbr).astype(bf16)
    og0, og1 = ogt[:, :_E], ogt[:, _E:]

    pos_j = jnp.concatenate(
        [jnp.dot(phi[0], og0, preferred_element_type=f32)
         + jnp.dot(plo[0], og0, preferred_element_type=f32),
         jnp.dot(phi[1], og1, preferred_element_type=f32)
         + jnp.dot(plo[1], og1, preferred_element_type=f32)], axis=1)
    post = phi.astype(f32) + plo.astype(f32)
    ctr = jnp.concatenate([post[0]] * _NBR + [post[1]] * _NBR, axis=1)
    rt = pos_j - ctr
    d = jnp.sqrt(jnp.sum(rt * rt, axis=0, keepdims=True) + 1e-10)

    mu = jax.lax.broadcasted_iota(jnp.int32, (_NRBF, 2 * _E), 0).astype(f32)
    mu = mu * (_CUTOFF / (_NRBF - 1))
    rbft = jnp.exp(-_GAMMA * (d - mu) ** 2)
    h1t = _ssp_fast(jnp.dot(wf1t_ref[...], rbft, preferred_element_type=f32)
                    + bf1_ref[...])
    wft = jnp.dot(wf2t_ref[...], h1t, preferred_element_type=f32) + bf2_ref[...]

    theta = jnp.minimum(d, _CUTOFF) * (jnp.pi / _CUTOFF)
    wft = wft * _fcut_poly(theta * theta)
    wf0, wf1 = wft[:, :_E], wft[:, _E:]

    for t in range(_NINT):
        vt = jnp.dot(wint_ref[t], xt, preferred_element_type=f32) + bin_ref[t]
        v16 = vt.astype(bf16)
        y0 = _lanefold(jnp.dot(v16[:, :_A], og0, preferred_element_type=f32)
                       * wf0, _A)
        y1 = _lanefold(jnp.dot(v16[:, _A:], og1, preferred_element_type=f32)
                       * wf1, _A)
        yt = jnp.concatenate([y0, y1], axis=1)
        yt = _ssp(jnp.dot(woutt_ref[t], yt, preferred_element_type=f32)
                  + bout_ref[t])
        xt = xt + yt

    ht = _ssp(jnp.dot(wo1t_ref[...], xt, preferred_element_type=f32)
              + bo1_ref[...])
    dost = jnp.dot(wo2t_ref[...], ht, preferred_element_type=f32) + bo2_ref[...]
    s = dost * amask
    out_ref[0] = jnp.concatenate(
        [jnp.sum(s[:, :_A], axis=1, keepdims=True),
         jnp.sum(s[:, _A:], axis=1, keepdims=True)], axis=1)


def kernel(positions, Z, neighbors, neighbor_mask, atom_mask, embedding,
           W_f1, b_f1, W_f2, b_f2, W_in, b_in, W_out, b_out,
           W_o1, b_o1, W_o2, b_o2):
    f32 = jnp.float32
    bf16 = jnp.bfloat16
    nbr = jnp.where(neighbor_mask > 0, neighbors.astype(jnp.int32), _A)
    nbr = nbr.transpose(0, 2, 1).reshape(_B // 2, 1, 2 * _E)
    z3 = Z.astype(jnp.int32).reshape(_B // 2, 1, 2 * _A)
    amask3 = atom_mask.astype(f32).reshape(_B // 2, 1, 2 * _A)
    post = positions.astype(f32).transpose(0, 2, 1)
    post_hi = post.astype(bf16).reshape(_B // 2, 2, 3, _A)
    post_lo = ((post - post.astype(bf16).astype(f32)).astype(bf16)
               .reshape(_B // 2, 2, 3, _A))

    def fullspec(x):
        return pl.BlockSpec(x.shape, lambda b: (0,) * x.ndim)

    weights = (embedding.T, W_f1.T, b_f1.reshape(_F, 1),
               W_f2.T, b_f2.reshape(_F, 1),
               W_in.transpose(0, 2, 1), b_in.reshape(_NINT, _F, 1),
               W_out.transpose(0, 2, 1), b_out.reshape(_NINT, _F, 1),
               W_o1.T, b_o1.reshape(64, 1),
               W_o2.T, b_o2.reshape(_NDOS, 1))

    out = pl.pallas_call(
        _schnet_body,
        grid=(_B // 2,),
        in_specs=[
            pl.BlockSpec((1, 2, 3, _A), lambda b: (b, 0, 0, 0)),
            pl.BlockSpec((1, 2, 3, _A), lambda b: (b, 0, 0, 0)),
            pl.BlockSpec((1, 1, 2 * _A), lambda b: (b, 0, 0)),
            pl.BlockSpec((1, 1, 2 * _E), lambda b: (b, 0, 0)),
            pl.BlockSpec((1, 1, 2 * _A), lambda b: (b, 0, 0)),
        ] + [fullspec(w) for w in weights],
        out_specs=pl.BlockSpec((1, _NDOS, 2), lambda b: (b, 0, 0)),
        out_shape=jax.ShapeDtypeStruct((_B // 2, _NDOS, 2), f32),
    )(post_hi, post_lo, z3, nbr, amask3, *weights)
    return out.transpose(0, 2, 1).reshape(_B, _NDOS)
